# baseline (speedup 1.0000x reference)
import functools

import jax
import jax.numpy as jnp
from jax import lax
from jax.experimental import pallas as pl
from jax.experimental.pallas import tpu as pltpu

_H = 224
_W = 224
_HW = _H * _W
_C = 64


def _conv_w(w):
    return w.transpose(2, 3, 1, 0).reshape(9 * w.shape[1], w.shape[0])


def _patches(r0, r1, r2):
    cols = []
    for r in (r0, r1, r2):
        for dx in range(3):
            cols.append(r[dx:dx + _W, :])
    return jnp.concatenate(cols, axis=1)


def _convA_body(x0, x1, x2, m, w, b, out, outm):
    p = _patches(x0[0, 0], x1[0, 0], x2[0, 0])
    y = jnp.dot(p, w[0], preferred_element_type=jnp.float32) + b[0]
    y0 = jax.nn.relu(y[:, :_C])
    out[0, 0] = jnp.concatenate([y0, y[:, _C:]], axis=1)
    outm[0, 0] = y0 * (1.0 - m[0, 0])


def _convE_body(x0, x1, x2, f1, w, b, out):
    p = _patches(x0[0, 0], x1[0, 0], x2[0, 0])
    y = jnp.dot(p, w[0], preferred_element_type=jnp.float32) + b[0]
    out[0, 0] = jax.nn.relu(y + f1[0, 0])


def _row_specs():
    def mk(dy):
        return pl.BlockSpec((1, 1, _W + 2, _C), lambda g, i, dy=dy: (g, i + dy, 0, 0))
    return [mk(0), mk(1), mk(2)]


def _conv01(x_pad, m1, w01, b01):
    G = x_pad.shape[0]
    grid = (G, _H)
    in_specs = _row_specs() + [
        pl.BlockSpec((1, 1, _W, 1), lambda g, i: (g, i, 0, 0)),
        pl.BlockSpec((1, 576, 2 * _C), lambda g, i: (g // 2, 0, 0)),
        pl.BlockSpec((1, 1, 2 * _C), lambda g, i: (g // 2, 0, 0)),
    ]
    out_specs = [
        pl.BlockSpec((1, 1, _W, 2 * _C), lambda g, i: (g, i, 0, 0)),
        pl.BlockSpec((1, 1, _W, _C), lambda g, i: (g, i, 0, 0)),
    ]
    return pl.pallas_call(
        _convA_body,
        grid=grid,
        in_specs=in_specs,
        out_specs=out_specs,
        out_shape=[
            jax.ShapeDtypeStruct((G, _H, _W, 2 * _C), jnp.float32),
            jax.ShapeDtypeStruct((G, _H, _W, _C), jnp.float32),
        ],
        compiler_params=pltpu.CompilerParams(
            dimension_semantics=("parallel", "arbitrary")),
    )(x_pad, x_pad, x_pad, m1, w01, b01)


def _conv2(x_pad, f1, w2, b2):
    G = x_pad.shape[0]
    grid = (G, _H)
    in_specs = _row_specs() + [
        pl.BlockSpec((1, 1, _W, _C), lambda g, i: (g, i, 0, 0)),
        pl.BlockSpec((1, 576, _C), lambda g, i: (g // 2, 0, 0)),
        pl.BlockSpec((1, 1, _C), lambda g, i: (g // 2, 0, 0)),
    ]
    out_specs = pl.BlockSpec((1, 1, _W, _C), lambda g, i: (g, i, 0, 0))
    return pl.pallas_call(
        _convE_body,
        grid=grid,
        in_specs=in_specs,
        out_specs=out_specs,
        out_shape=jax.ShapeDtypeStruct((G, _H, _W, _C), jnp.float32),
        compiler_params=pltpu.CompilerParams(
            dimension_semantics=("parallel", "arbitrary")),
    )(x_pad, x_pad, x_pad, f1, w2, b2)


def _pad_hw(x):
    return jnp.pad(x, ((0, 0), (1, 1), (1, 1), (0, 0)))


def kernel(d_feat, r_feat, spoints, sidxs, nnidxs, masks, nsamples,
           d_conv0_W, d_conv0_b, d_conv1_W, d_conv1_b, d_conv2_W, d_conv2_b,
           r_conv0_W, r_conv0_b, r_conv1_W, r_conv1_b, r_conv2_W, r_conv2_b,
           d_mlp_W1, d_mlp_b1, d_mlp_W2, d_mlp_b2,
           r_mlp_W1, r_mlp_b1, r_mlp_W2, r_mlp_b2,
           d_bias, r_bias):
    B = d_feat.shape[0]
    G = 2 * B
    k = nnidxs.shape[2]
    ns = sidxs.shape[1]

    x = jnp.concatenate([d_feat.transpose(0, 2, 3, 1),
                         r_feat.transpose(0, 2, 3, 1)], axis=0)
    x_pad = _pad_hw(x)
    m = masks.astype(jnp.float32).transpose(0, 2, 3, 1)
    m1 = jnp.concatenate([m, m], axis=0)

    w01 = jnp.stack([
        jnp.concatenate([_conv_w(d_conv0_W), _conv_w(d_conv1_W)], axis=1),
        jnp.concatenate([_conv_w(r_conv0_W), _conv_w(r_conv1_W)], axis=1),
    ])
    b01 = jnp.stack([jnp.concatenate([d_conv0_b, d_conv1_b])[None, :],
                     jnp.concatenate([r_conv0_b, r_conv1_b])[None, :]])

    y01, y0m = _conv01(x_pad, m1, w01, b01)
    feat0 = y01[..., :_C]
    feat1 = y01[..., _C:]

    f0 = feat0.reshape(G, _HW, _C)
    d_f0, r_f0 = f0[:B], f0[B:]
    gat = jax.vmap(lambda t, i: t[i])
    d_sf = gat(d_f0, sidxs)
    r_sf = gat(r_f0, sidxs)
    sp = spoints.transpose(0, 2, 1)
    nnp = gat(sp, nnidxs)
    d_nn = gat(d_sf, nnidxs)
    r_nn = gat(r_sf, nnidxs)
    feats = jnp.concatenate([d_nn - d_sf[:, :, None, :],
                             r_nn - r_sf[:, :, None, :],
                             nnp - sp[:, :, None, :]], axis=-1)

    def mlp_attn(W1, b1, W2, b2):
        h = feats @ W1.T + b1
        h = jnp.where(h >= 0, h, 0.2 * h)
        logit = (h @ W2.T + b2)[..., 0]
        return jax.nn.softmax(logit, axis=-1)

    d_attn = mlp_attn(d_mlp_W1, d_mlp_b1, d_mlp_W2, d_mlp_b2)
    r_attn = mlp_attn(r_mlp_W1, r_mlp_b1, r_mlp_W2, r_mlp_b2)
    d_agg = jnp.einsum('bsk,bskc->bsc', d_attn, d_nn) + d_bias
    r_agg = jnp.einsum('bsk,bskc->bsc', r_attn, r_nn) + r_bias

    f0m = y0m.reshape(G, _HW, _C)
    def blend(base_m, agg, i):
        new = jnp.zeros((_HW, _C), jnp.float32).at[i].set(agg)
        return base_m + new
    d_upd = jax.vmap(blend)(f0m[:B], d_agg, sidxs)
    r_upd = jax.vmap(blend)(f0m[B:], r_agg, sidxs)
    upd = jnp.concatenate([d_upd, r_upd], axis=0).reshape(G, _H, _W, _C)

    w2 = jnp.stack([_conv_w(d_conv2_W), _conv_w(r_conv2_W)])
    b2 = jnp.stack([d_conv2_b[None, :], r_conv2_b[None, :]])
    y = _conv2(_pad_hw(upd), feat1, w2, b2)

    out = y.transpose(0, 3, 1, 2)
    return out[:B], out[B:]


# speedup vs baseline: 1.4789x; 1.4789x over previous
import functools

import jax
import jax.numpy as jnp
from jax import lax
from jax.experimental import pallas as pl
from jax.experimental.pallas import tpu as pltpu
from jax.experimental.pallas import tpu_sc as plsc

_H = 224
_W = 224
_HW = _H * _W
_C = 64
_NS = 4096
_K = 9
_NW = 32
_NNB = _NS * _K


def _conv_w(w):
    return w.transpose(2, 3, 1, 0).reshape(9 * w.shape[1], w.shape[0])


def _patches(r0, r1, r2):
    cols = []
    for r in (r0, r1, r2):
        for dx in range(3):
            cols.append(r[dx:dx + _W, :])
    return jnp.concatenate(cols, axis=1)


def _convA_body(x0, x1, x2, m, w, b, out, outm):
    p = _patches(x0[0, 0], x1[0, 0], x2[0, 0])
    y = jnp.dot(p, w[0], preferred_element_type=jnp.float32) + b[0]
    y0 = jax.nn.relu(y[:, :_C])
    out[0, 0] = jnp.concatenate([y0, y[:, _C:]], axis=1)
    outm[0, 0] = y0 * (1.0 - m[0, 0])


def _convE_body(x0, x1, x2, f1, w, b, out):
    p = _patches(x0[0, 0], x1[0, 0], x2[0, 0])
    y = jnp.dot(p, w[0], preferred_element_type=jnp.float32) + b[0]
    out[0, 0] = jax.nn.relu(y + f1[0, 0])


def _row_specs():
    def mk(dy):
        return pl.BlockSpec((1, 1, _W + 2, _C), lambda g, i, dy=dy: (g, i + dy, 0, 0))
    return [mk(0), mk(1), mk(2)]


def _conv01(x_pad, m1, w01, b01):
    G = x_pad.shape[0]
    grid = (G, _H)
    in_specs = _row_specs() + [
        pl.BlockSpec((1, 1, _W, 1), lambda g, i: (g, i, 0, 0)),
        pl.BlockSpec((1, 576, 2 * _C), lambda g, i: (g // 2, 0, 0)),
        pl.BlockSpec((1, 1, 2 * _C), lambda g, i: (g // 2, 0, 0)),
    ]
    out_specs = [
        pl.BlockSpec((1, 1, _W, 2 * _C), lambda g, i: (g, i, 0, 0)),
        pl.BlockSpec((1, 1, _W, _C), lambda g, i: (g, i, 0, 0)),
    ]
    return pl.pallas_call(
        _convA_body,
        grid=grid,
        in_specs=in_specs,
        out_specs=out_specs,
        out_shape=[
            jax.ShapeDtypeStruct((G, _H, _W, 2 * _C), jnp.float32),
            jax.ShapeDtypeStruct((G, _H, _W, _C), jnp.float32),
        ],
        compiler_params=pltpu.CompilerParams(
            dimension_semantics=("parallel", "arbitrary")),
    )(x_pad, x_pad, x_pad, m1, w01, b01)


def _conv2(x_pad, f1, w2, b2):
    G = x_pad.shape[0]
    grid = (G, _H)
    in_specs = _row_specs() + [
        pl.BlockSpec((1, 1, _W, _C), lambda g, i: (g, i, 0, 0)),
        pl.BlockSpec((1, 576, _C), lambda g, i: (g // 2, 0, 0)),
        pl.BlockSpec((1, 1, _C), lambda g, i: (g // 2, 0, 0)),
    ]
    out_specs = pl.BlockSpec((1, 1, _W, _C), lambda g, i: (g, i, 0, 0))
    return pl.pallas_call(
        _convE_body,
        grid=grid,
        in_specs=in_specs,
        out_specs=out_specs,
        out_shape=jax.ShapeDtypeStruct((G, _H, _W, _C), jnp.float32),
        compiler_params=pltpu.CompilerParams(
            dimension_semantics=("parallel", "arbitrary")),
    )(x_pad, x_pad, x_pad, f1, w2, b2)


def _pad_hw(x):
    return jnp.pad(x, ((0, 0), (1, 1), (1, 1), (0, 0)))


def _sc_gather(f0, f0m, sidxs, nnidx, spts):
    mesh = plsc.VectorSubcoreMesh(core_axis_name="c", subcore_axis_name="s")
    B = sidxs.shape[0]
    nchunk = _NNB // _NW // 128

    out_type = [
        jax.ShapeDtypeStruct((B * _NNB, _C), jnp.float32),
        jax.ShapeDtypeStruct((B * _NNB, _C), jnp.float32),
        jax.ShapeDtypeStruct((B * _NNB, 16), jnp.float32),
        jax.ShapeDtypeStruct((B * _NS, _C), jnp.float32),
        jax.ShapeDtypeStruct((B * _NS, _C), jnp.float32),
        jax.ShapeDtypeStruct((B * _NS, _C), jnp.float32),
        jax.ShapeDtypeStruct((B * _NS, _C), jnp.float32),
    ]

    @functools.partial(
        pl.kernel, mesh=mesh, out_type=out_type,
        compiler_params=pltpu.CompilerParams(needs_layout_passes=False, use_tc_tiling_on_sc=False),
        scratch_types=[
            pltpu.VMEM((_NS,), jnp.int32),
            pltpu.VMEM((128,), jnp.int32),
            pltpu.VMEM((128,), jnp.int32),
            pltpu.VMEM((128,), jnp.int32),
            pltpu.VMEM((128,), jnp.int32),
            pltpu.VMEM((128, _C), jnp.float32),
            pltpu.VMEM((128, _C), jnp.float32),
            pltpu.VMEM((128, 16), jnp.float32),
            pltpu.SemaphoreType.DMA,
            pltpu.SemaphoreType.DMA,
            pltpu.SemaphoreType.DMA,
        ],
    )
    def body(f0_h, f0m_h, sidx_h, nn_h, sp_h,
             dnn_h, rnn_h, pnn_h, sfd_h, sfr_h, msfd_h, msfr_h,
             sidx_v, nnq_v, idxd_v, idxr_v, idxp_v, bufd, bufr, bufp,
             sem1, sem2, sem3):
        wid = lax.axis_index("s") * 2 + lax.axis_index("c")
        for b in range(B):
            offd = b * _HW
            offr = (2 + b) * _HW
            pltpu.sync_copy(sidx_h.at[b], sidx_v)

            def nn_chunk(j, _):
                q = wid * nchunk + j
                row0 = q * 128
                pltpu.sync_copy(nn_h.at[b, pl.ds(row0, 128)], nnq_v)

                def compose(t, _):
                    i16 = nnq_v[pl.ds(t * 16, 16)]
                    s16 = plsc.load_gather(sidx_v, [i16])
                    idxd_v[pl.ds(t * 16, 16)] = s16 + offd
                    idxr_v[pl.ds(t * 16, 16)] = s16 + offr
                    idxp_v[pl.ds(t * 16, 16)] = i16 + b * _NS
                    return 0
                lax.fori_loop(0, 8, compose, 0)
                c1 = pltpu.async_copy(f0_h.at[idxd_v], bufd, sem1)
                c2 = pltpu.async_copy(f0_h.at[idxr_v], bufr, sem2)
                c3 = pltpu.async_copy(sp_h.at[idxp_v], bufp, sem3)
                c1.wait(); c2.wait(); c3.wait()
                o0 = b * _NNB + row0
                pltpu.sync_copy(bufd, dnn_h.at[pl.ds(o0, 128)])
                pltpu.sync_copy(bufr, rnn_h.at[pl.ds(o0, 128)])
                pltpu.sync_copy(bufp, pnn_h.at[pl.ds(o0, 128)])
                return 0
            lax.fori_loop(0, nchunk, nn_chunk, 0)

            def compose_s(t, _):
                s16 = sidx_v[pl.ds(wid * 128 + t * 16, 16)]
                idxd_v[pl.ds(t * 16, 16)] = s16 + offd
                idxr_v[pl.ds(t * 16, 16)] = s16 + offr
                return 0
            lax.fori_loop(0, 8, compose_s, 0)
            o0 = b * _NS + wid * 128
            c1 = pltpu.async_copy(f0_h.at[idxd_v], bufd, sem1)
            c2 = pltpu.async_copy(f0_h.at[idxr_v], bufr, sem2)
            c1.wait(); c2.wait()
            pltpu.sync_copy(bufd, sfd_h.at[pl.ds(o0, 128)])
            pltpu.sync_copy(bufr, sfr_h.at[pl.ds(o0, 128)])
            c1 = pltpu.async_copy(f0m_h.at[idxd_v], bufd, sem1)
            c2 = pltpu.async_copy(f0m_h.at[idxr_v], bufr, sem2)
            c1.wait(); c2.wait()
            pltpu.sync_copy(bufd, msfd_h.at[pl.ds(o0, 128)])
            pltpu.sync_copy(bufr, msfr_h.at[pl.ds(o0, 128)])

    return body(f0, f0m, sidxs, nnidx, spts)


def _sc_scatter(base, vals, sidxs, winof):
    mesh = plsc.VectorSubcoreMesh(core_axis_name="c", subcore_axis_name="s")
    B = sidxs.shape[0]
    rows_per_core = B * _HW
    rows_per_tile = rows_per_core // 16
    samp_per_tile = _NS // 16

    @functools.partial(
        pl.kernel, mesh=mesh,
        out_type=jax.ShapeDtypeStruct((2 * B * _HW, _C), jnp.float32),
        compiler_params=pltpu.CompilerParams(needs_layout_passes=False, use_tc_tiling_on_sc=False),
        scratch_types=[
            pltpu.VMEM((128,), jnp.int32),
            pltpu.VMEM((128,), jnp.int32),
            pltpu.VMEM((128,), jnp.int32),
            pltpu.VMEM((128,), jnp.int32),
            pltpu.VMEM((128, _C), jnp.float32),
            pltpu.SemaphoreType.DMA,
        ],
    )
    def body(base_h, vals_h, sidx_h, win_h, out_h,
             sidq_v, winq_v, widx_v, tidx_v, bufv, sem):
        cid = lax.axis_index("c")
        sid = lax.axis_index("s")
        row0 = cid * rows_per_core + sid * rows_per_tile
        pltpu.sync_copy(base_h.at[pl.ds(row0, rows_per_tile)],
                        out_h.at[pl.ds(row0, rows_per_tile)])
        plsc.subcore_barrier()
        for b in range(B):
            voff = cid * (B * _NS) + b * _NS
            toff = (2 * cid + b) * _HW
            for u in range(samp_per_tile // 128):
                s0 = sid * samp_per_tile + u * 128
                pltpu.sync_copy(sidx_h.at[b, pl.ds(s0, 128)], sidq_v)
                pltpu.sync_copy(win_h.at[b, pl.ds(s0, 128)], winq_v)

                def compose(t, _):
                    widx_v[pl.ds(t * 16, 16)] = winq_v[pl.ds(t * 16, 16)] + voff
                    tidx_v[pl.ds(t * 16, 16)] = sidq_v[pl.ds(t * 16, 16)] + toff
                    return 0
                lax.fori_loop(0, 8, compose, 0)
                pltpu.async_copy(vals_h.at[widx_v], bufv, sem).wait()
                pltpu.async_copy(bufv, out_h.at[tidx_v], sem).wait()

    return body(base, vals, sidxs, winof)


def _attn_body(dnn, rnn, pnn, sfd, sfr, sp, msfd, msfr,
               w1, b1, w2, bd, br, outd, outr):
    d3 = dnn[...]
    r3 = rnn[...]
    p3 = pnn[...]
    xs = jnp.concatenate([sfd[...], sfr[...], sp[...]], axis=1)
    base = jnp.dot(xs, w1[...], preferred_element_type=jnp.float32)
    dlogs, rlogs = [], []
    for kk in range(_K):
        xk = jnp.concatenate([d3[:, kk, :], r3[:, kk, :], p3[:, kk, :]],
                             axis=1)
        g = jnp.dot(xk, w1[...], preferred_element_type=jnp.float32)
        pre = g - base + b1[...]
        h = jnp.where(pre >= 0, pre, 0.2 * pre)
        prod = h * w2[...]
        dlogs.append(jnp.sum(prod[:, :128], axis=1, keepdims=True))
        rlogs.append(jnp.sum(prod[:, 128:], axis=1, keepdims=True))
    dlog = jnp.concatenate(dlogs, axis=1)
    rlog = jnp.concatenate(rlogs, axis=1)

    def soft(x):
        m = jnp.max(x, axis=1, keepdims=True)
        e = jnp.exp(x - m)
        return e / jnp.sum(e, axis=1, keepdims=True)
    dattn = soft(dlog)
    rattn = soft(rlog)
    accd = msfd[...] + bd[...]
    accr = msfr[...] + br[...]
    for kk in range(_K):
        accd = accd + dattn[:, kk:kk + 1] * d3[:, kk, :]
        accr = accr + rattn[:, kk:kk + 1] * r3[:, kk, :]
    outd[...] = accd
    outr[...] = accr


def _attn(dnn3, rnn3, pnn3, sfd, sfr, sp, msfd, msfr, w1, b1, w2, bd, br):
    S = sfd.shape[0]
    SB = 512
    grid = (S // SB,)
    bs2 = lambda c: pl.BlockSpec((SB, c), lambda i: (i, 0))
    bs3 = lambda c: pl.BlockSpec((SB, _K, c), lambda i: (i, 0, 0))
    full = lambda a, b: pl.BlockSpec((a, b), lambda i: (0, 0))
    return pl.pallas_call(
        _attn_body,
        grid=grid,
        in_specs=[bs3(_C), bs3(_C), bs3(16), bs2(_C), bs2(_C), bs2(16),
                  bs2(_C), bs2(_C),
                  full(144, 256), full(1, 256), full(1, 256),
                  full(1, _C), full(1, _C)],
        out_specs=[bs2(_C), bs2(_C)],
        out_shape=[jax.ShapeDtypeStruct((S, _C), jnp.float32),
                   jax.ShapeDtypeStruct((S, _C), jnp.float32)],
        compiler_params=pltpu.CompilerParams(
            dimension_semantics=("arbitrary",)),
    )(dnn3, rnn3, pnn3, sfd, sfr, sp, msfd, msfr, w1, b1, w2, bd, br)


def kernel(d_feat, r_feat, spoints, sidxs, nnidxs, masks, nsamples,
           d_conv0_W, d_conv0_b, d_conv1_W, d_conv1_b, d_conv2_W, d_conv2_b,
           r_conv0_W, r_conv0_b, r_conv1_W, r_conv1_b, r_conv2_W, r_conv2_b,
           d_mlp_W1, d_mlp_b1, d_mlp_W2, d_mlp_b2,
           r_mlp_W1, r_mlp_b1, r_mlp_W2, r_mlp_b2,
           d_bias, r_bias):
    B = d_feat.shape[0]
    G = 2 * B
    k = nnidxs.shape[2]
    ns = sidxs.shape[1]

    x = jnp.concatenate([d_feat.transpose(0, 2, 3, 1),
                         r_feat.transpose(0, 2, 3, 1)], axis=0)
    x_pad = _pad_hw(x)
    m = masks.astype(jnp.float32).transpose(0, 2, 3, 1)
    m1 = jnp.concatenate([m, m], axis=0)

    w01 = jnp.stack([
        jnp.concatenate([_conv_w(d_conv0_W), _conv_w(d_conv1_W)], axis=1),
        jnp.concatenate([_conv_w(r_conv0_W), _conv_w(r_conv1_W)], axis=1),
    ])
    b01 = jnp.stack([jnp.concatenate([d_conv0_b, d_conv1_b])[None, :],
                     jnp.concatenate([r_conv0_b, r_conv1_b])[None, :]])

    y01, y0m = _conv01(x_pad, m1, w01, b01)
    feat0 = y01[..., :_C]
    feat1 = y01[..., _C:]

    f0_flat = feat0.reshape(G * _HW, _C)
    f0m_flat = y0m.reshape(G * _HW, _C)
    sidx32 = sidxs.astype(jnp.int32)
    nn_flat = nnidxs.astype(jnp.int32).reshape(B, ns * k)
    spts = jnp.pad(spoints.transpose(0, 2, 1),
                   ((0, 0), (0, 0), (0, 13))).reshape(B * ns, 16)

    (d_nn, r_nn, p_nn, sfd, sfr, msfd, msfr) = _sc_gather(
        f0_flat, f0m_flat, sidx32, nn_flat, spts)

    w1 = jnp.zeros((144, 256), jnp.float32)
    w1 = w1.at[:131, :65].set(d_mlp_W1.T).at[:131, 128:193].set(r_mlp_W1.T)
    b1 = jnp.zeros((1, 256), jnp.float32)
    b1 = b1.at[0, :65].set(d_mlp_b1).at[0, 128:193].set(r_mlp_b1)
    w2 = jnp.zeros((1, 256), jnp.float32)
    w2 = w2.at[0, :65].set(d_mlp_W2[0]).at[0, 128:193].set(r_mlp_W2[0])

    d_rows, r_rows = _attn(
        d_nn.reshape(B * ns, k, _C), r_nn.reshape(B * ns, k, _C),
        p_nn.reshape(B * ns, k, 16), sfd, sfr, spts, msfd, msfr,
        w1, b1, w2, d_bias[None, :], r_bias[None, :])

    iot = jnp.arange(ns, dtype=jnp.int32)
    maxi = jax.vmap(lambda i: jnp.zeros((_HW,), jnp.int32).at[i].max(iot))(
        sidx32)
    winof = jax.vmap(lambda m, i: m[i])(maxi, sidx32)

    vals = jnp.concatenate([d_rows, r_rows], axis=0)
    upd_flat = _sc_scatter(f0m_flat, vals, sidx32, winof)
    upd = upd_flat.reshape(G, _H, _W, _C)

    w2 = jnp.stack([_conv_w(d_conv2_W), _conv_w(r_conv2_W)])
    b2 = jnp.stack([d_conv2_b[None, :], r_conv2_b[None, :]])
    y = _conv2(_pad_hw(upd), feat1, w2, b2)

    out = y.transpose(0, 3, 1, 2)
    return out[:B], out[B:]


# speedup vs baseline: 2.4004x; 1.6231x over previous
import functools

import jax
import jax.numpy as jnp
from jax import lax
from jax.experimental import pallas as pl
from jax.experimental.pallas import tpu as pltpu
from jax.experimental.pallas import tpu_sc as plsc

_H = 224
_W = 224
_HW = _H * _W
_C = 64
_NS = 4096
_K = 9
_NW = 32
_NNB = _NS * _K


def _conv_w(w):
    return w.transpose(2, 3, 1, 0).reshape(9 * w.shape[1], w.shape[0])


def _patches(r0, r1, r2):
    cols = []
    for r in (r0, r1, r2):
        for dx in range(3):
            cols.append(r[dx:dx + _W, :])
    return jnp.concatenate(cols, axis=1)


def _convA_body(x0, x1, x2, m, w, b, out, outm):
    p = _patches(x0[0, 0], x1[0, 0], x2[0, 0])
    y = jnp.dot(p, w[0], preferred_element_type=jnp.float32) + b[0]
    y0 = jax.nn.relu(y[:, :_C])
    out[0, 0] = jnp.concatenate([y0, y[:, _C:]], axis=1)
    outm[0, 0] = y0 * (1.0 - m[0, 0])


def _convE_body(x0, x1, x2, f1, w, b, out):
    p = _patches(x0[0, 0], x1[0, 0], x2[0, 0])
    y = jnp.dot(p, w[0], preferred_element_type=jnp.float32) + b[0]
    out[0, 0] = jax.nn.relu(y + f1[0, 0])


def _row_specs():
    def mk(dy):
        return pl.BlockSpec((1, 1, _W + 2, _C), lambda g, i, dy=dy: (g, i + dy, 0, 0))
    return [mk(0), mk(1), mk(2)]


def _conv01(x_pad, m1, w01, b01):
    G = x_pad.shape[0]
    grid = (G, _H)
    in_specs = _row_specs() + [
        pl.BlockSpec((1, 1, _W, 1), lambda g, i: (g, i, 0, 0)),
        pl.BlockSpec((1, 576, 2 * _C), lambda g, i: (g // 2, 0, 0)),
        pl.BlockSpec((1, 1, 2 * _C), lambda g, i: (g // 2, 0, 0)),
    ]
    out_specs = [
        pl.BlockSpec((1, 1, _W, 2 * _C), lambda g, i: (g, i, 0, 0)),
        pl.BlockSpec((1, 1, _W, _C), lambda g, i: (g, i, 0, 0)),
    ]
    return pl.pallas_call(
        _convA_body,
        grid=grid,
        in_specs=in_specs,
        out_specs=out_specs,
        out_shape=[
            jax.ShapeDtypeStruct((G, _H, _W, 2 * _C), jnp.float32),
            jax.ShapeDtypeStruct((G, _H, _W, _C), jnp.float32),
        ],
        compiler_params=pltpu.CompilerParams(
            dimension_semantics=("parallel", "arbitrary")),
    )(x_pad, x_pad, x_pad, m1, w01, b01)


def _conv2(x_pad, f1, w2, b2):
    G = x_pad.shape[0]
    grid = (G, _H)
    in_specs = _row_specs() + [
        pl.BlockSpec((1, 1, _W, _C), lambda g, i: (g, i, 0, 0)),
        pl.BlockSpec((1, 576, _C), lambda g, i: (g // 2, 0, 0)),
        pl.BlockSpec((1, 1, _C), lambda g, i: (g // 2, 0, 0)),
    ]
    out_specs = pl.BlockSpec((1, 1, _W, _C), lambda g, i: (g, i, 0, 0))
    return pl.pallas_call(
        _convE_body,
        grid=grid,
        in_specs=in_specs,
        out_specs=out_specs,
        out_shape=jax.ShapeDtypeStruct((G, _H, _W, _C), jnp.float32),
        compiler_params=pltpu.CompilerParams(
            dimension_semantics=("parallel", "arbitrary")),
    )(x_pad, x_pad, x_pad, f1, w2, b2)


def _pad_hw(x):
    return jnp.pad(x, ((0, 0), (1, 1), (1, 1), (0, 0)))


def _sc_gather(f0, f0m, sidxs, nnidx, spts):
    mesh = plsc.VectorSubcoreMesh(core_axis_name="c", subcore_axis_name="s")
    B = sidxs.shape[0]
    nchunk = _NNB // _NW // 128

    out_type = [
        jax.ShapeDtypeStruct((B * _NNB, _C), jnp.float32),
        jax.ShapeDtypeStruct((B * _NNB, _C), jnp.float32),
        jax.ShapeDtypeStruct((B * _NNB, 16), jnp.float32),
        jax.ShapeDtypeStruct((B * _NS, _C), jnp.float32),
        jax.ShapeDtypeStruct((B * _NS, _C), jnp.float32),
        jax.ShapeDtypeStruct((B * _NS, _C), jnp.float32),
        jax.ShapeDtypeStruct((B * _NS, _C), jnp.float32),
    ]

    @functools.partial(
        pl.kernel, mesh=mesh, out_type=out_type,
        compiler_params=pltpu.CompilerParams(needs_layout_passes=False, use_tc_tiling_on_sc=False),
        scratch_types=[
            pltpu.VMEM((_NS,), jnp.int32),
            pltpu.VMEM((2, 128), jnp.int32),
            pltpu.VMEM((2, 128), jnp.int32),
            pltpu.VMEM((2, 128), jnp.int32),
            pltpu.VMEM((2, 128), jnp.int32),
            pltpu.VMEM((2, 128, _C), jnp.float32),
            pltpu.VMEM((2, 128, _C), jnp.float32),
            pltpu.VMEM((2, 128, 16), jnp.float32),
            pltpu.SemaphoreType.DMA((2, 3)),
            pltpu.SemaphoreType.DMA((2, 3)),
        ],
    )
    def body(f0_h, f0m_h, sidx_h, nn_h, sp_h,
             dnn_h, rnn_h, pnn_h, sfd_h, sfr_h, msfd_h, msfr_h,
             sidx_v, nnq_v, idxd_v, idxr_v, idxp_v, bufd, bufr, bufp,
             gsems, osems):
        wid = lax.axis_index("s") * 2 + lax.axis_index("c")
        for b in range(B):
            offd = b * _HW
            offr = (2 + b) * _HW
            pltpu.sync_copy(sidx_h.at[b], sidx_v)

            gathers = [None, None]

            def stage(j, p):
                q = wid * nchunk + j
                row0 = q * 128
                pltpu.sync_copy(nn_h.at[b, pl.ds(row0, 128)], nnq_v.at[p])

                def compose(t, _):
                    i16 = nnq_v[p, pl.ds(t * 16, 16)]
                    s16 = plsc.load_gather(sidx_v, [i16])
                    idxd_v[p, pl.ds(t * 16, 16)] = s16 + offd
                    idxr_v[p, pl.ds(t * 16, 16)] = s16 + offr
                    idxp_v[p, pl.ds(t * 16, 16)] = i16 + b * _NS
                    return 0
                lax.fori_loop(0, 8, compose, 0)
                return (
                    pltpu.async_copy(f0_h.at[idxd_v.at[p]], bufd.at[p],
                                     gsems.at[p, 0]),
                    pltpu.async_copy(f0_h.at[idxr_v.at[p]], bufr.at[p],
                                     gsems.at[p, 1]),
                    pltpu.async_copy(sp_h.at[idxp_v.at[p]], bufp.at[p],
                                     gsems.at[p, 2]),
                )

            outs = [None, None]

            def writeback(j, p):
                for c in gathers[p]:
                    c.wait()
                o0 = b * _NNB + (wid * nchunk + j) * 128
                outs[p] = (
                    pltpu.async_copy(bufd.at[p], dnn_h.at[pl.ds(o0, 128)],
                                     osems.at[p, 0]),
                    pltpu.async_copy(bufr.at[p], rnn_h.at[pl.ds(o0, 128)],
                                     osems.at[p, 1]),
                    pltpu.async_copy(bufp.at[p], pnn_h.at[pl.ds(o0, 128)],
                                     osems.at[p, 2]),
                )

            for j in range(nchunk):
                p = j % 2
                if j >= 2:
                    for c in outs[p]:
                        c.wait()
                gathers[p] = stage(j, p)
                if j >= 1:
                    writeback(j - 1, 1 - p)
            writeback(nchunk - 1, (nchunk - 1) % 2)
            for pp in (0, 1):
                for c in outs[pp]:
                    c.wait()

            def compose_s(t, _):
                s16 = sidx_v[pl.ds(wid * 128 + t * 16, 16)]
                idxd_v[0, pl.ds(t * 16, 16)] = s16 + offd
                idxr_v[0, pl.ds(t * 16, 16)] = s16 + offr
                return 0
            lax.fori_loop(0, 8, compose_s, 0)
            o0 = b * _NS + wid * 128
            c1 = pltpu.async_copy(f0_h.at[idxd_v.at[0]], bufd.at[0],
                                  gsems.at[0, 0])
            c2 = pltpu.async_copy(f0_h.at[idxr_v.at[0]], bufr.at[0],
                                  gsems.at[0, 1])
            c3 = pltpu.async_copy(f0m_h.at[idxd_v.at[0]], bufd.at[1],
                                  gsems.at[1, 0])
            c4 = pltpu.async_copy(f0m_h.at[idxr_v.at[0]], bufr.at[1],
                                  gsems.at[1, 1])
            c1.wait(); c2.wait(); c3.wait(); c4.wait()
            w1 = pltpu.async_copy(bufd.at[0], sfd_h.at[pl.ds(o0, 128)],
                                  osems.at[0, 0])
            w2 = pltpu.async_copy(bufr.at[0], sfr_h.at[pl.ds(o0, 128)],
                                  osems.at[0, 1])
            w3 = pltpu.async_copy(bufd.at[1], msfd_h.at[pl.ds(o0, 128)],
                                  osems.at[1, 0])
            w4 = pltpu.async_copy(bufr.at[1], msfr_h.at[pl.ds(o0, 128)],
                                  osems.at[1, 1])
            w1.wait(); w2.wait(); w3.wait(); w4.wait()

    return body(f0, f0m, sidxs, nnidx, spts)


def _sc_scatter(base, vals, sidxs, winof):
    mesh = plsc.VectorSubcoreMesh(core_axis_name="c", subcore_axis_name="s")
    B = sidxs.shape[0]
    samp_per_tile = _NS // 16

    def stateful(refs):
        out_h, vals_h, sidx_h, win_h = refs

        @pl.core_map(
            mesh,
            compiler_params=pltpu.CompilerParams(
                needs_layout_passes=False, use_tc_tiling_on_sc=False),
            scratch_shapes=[
                pltpu.VMEM((128,), jnp.int32),
                pltpu.VMEM((128,), jnp.int32),
                pltpu.VMEM((128,), jnp.int32),
                pltpu.VMEM((128,), jnp.int32),
                pltpu.VMEM((128, _C), jnp.float32),
                pltpu.SemaphoreType.DMA,
            ],
        )
        def _(sidq_v, winq_v, widx_v, tidx_v, bufv, sem):
            cid = lax.axis_index("c")
            sid = lax.axis_index("s")
            for b in range(B):
                voff = cid * (B * _NS) + b * _NS
                toff = (2 * cid + b) * _HW
                for u in range(samp_per_tile // 128):
                    s0 = sid * samp_per_tile + u * 128
                    pltpu.sync_copy(sidx_h.at[b, pl.ds(s0, 128)], sidq_v)
                    pltpu.sync_copy(win_h.at[b, pl.ds(s0, 128)], winq_v)

                    def compose(t, _):
                        widx_v[pl.ds(t * 16, 16)] = (
                            winq_v[pl.ds(t * 16, 16)] + voff)
                        tidx_v[pl.ds(t * 16, 16)] = (
                            sidq_v[pl.ds(t * 16, 16)] + toff)
                        return 0
                    lax.fori_loop(0, 8, compose, 0)
                    pltpu.async_copy(vals_h.at[widx_v], bufv, sem).wait()
                    pltpu.async_copy(bufv, out_h.at[tidx_v], sem).wait()

    out, _, _, _ = pl.run_state(stateful)((base, vals, sidxs, winof))
    return out


def _attn_body(dnn, rnn, pnn, sfd, sfr, sp, msfd, msfr,
               w1, b1, w2, bd, br, outd, outr):
    d3 = dnn[...]
    r3 = rnn[...]
    p3 = pnn[...]
    xs = jnp.concatenate([sfd[...], sfr[...], sp[...]], axis=1)
    base = jnp.dot(xs, w1[...], preferred_element_type=jnp.float32)
    dlogs, rlogs = [], []
    for kk in range(_K):
        xk = jnp.concatenate([d3[:, kk, :], r3[:, kk, :], p3[:, kk, :]],
                             axis=1)
        g = jnp.dot(xk, w1[...], preferred_element_type=jnp.float32)
        pre = g - base + b1[...]
        h = jnp.where(pre >= 0, pre, 0.2 * pre)
        prod = h * w2[...]
        dlogs.append(jnp.sum(prod[:, :128], axis=1, keepdims=True))
        rlogs.append(jnp.sum(prod[:, 128:], axis=1, keepdims=True))
    dlog = jnp.concatenate(dlogs, axis=1)
    rlog = jnp.concatenate(rlogs, axis=1)

    def soft(x):
        m = jnp.max(x, axis=1, keepdims=True)
        e = jnp.exp(x - m)
        return e / jnp.sum(e, axis=1, keepdims=True)
    dattn = soft(dlog)
    rattn = soft(rlog)
    accd = msfd[...] + bd[...]
    accr = msfr[...] + br[...]
    for kk in range(_K):
        accd = accd + dattn[:, kk:kk + 1] * d3[:, kk, :]
        accr = accr + rattn[:, kk:kk + 1] * r3[:, kk, :]
    outd[...] = accd
    outr[...] = accr


def _attn(dnn3, rnn3, pnn3, sfd, sfr, sp, msfd, msfr, w1, b1, w2, bd, br):
    S = sfd.shape[0]
    SB = 512
    grid = (S // SB,)
    bs2 = lambda c: pl.BlockSpec((SB, c), lambda i: (i, 0))
    bs3 = lambda c: pl.BlockSpec((SB, _K, c), lambda i: (i, 0, 0))
    full = lambda a, b: pl.BlockSpec((a, b), lambda i: (0, 0))
    return pl.pallas_call(
        _attn_body,
        grid=grid,
        in_specs=[bs3(_C), bs3(_C), bs3(16), bs2(_C), bs2(_C), bs2(16),
                  bs2(_C), bs2(_C),
                  full(144, 256), full(1, 256), full(1, 256),
                  full(1, _C), full(1, _C)],
        out_specs=[bs2(_C), bs2(_C)],
        out_shape=[jax.ShapeDtypeStruct((S, _C), jnp.float32),
                   jax.ShapeDtypeStruct((S, _C), jnp.float32)],
        compiler_params=pltpu.CompilerParams(
            dimension_semantics=("arbitrary",)),
    )(dnn3, rnn3, pnn3, sfd, sfr, sp, msfd, msfr, w1, b1, w2, bd, br)


def kernel(d_feat, r_feat, spoints, sidxs, nnidxs, masks, nsamples,
           d_conv0_W, d_conv0_b, d_conv1_W, d_conv1_b, d_conv2_W, d_conv2_b,
           r_conv0_W, r_conv0_b, r_conv1_W, r_conv1_b, r_conv2_W, r_conv2_b,
           d_mlp_W1, d_mlp_b1, d_mlp_W2, d_mlp_b2,
           r_mlp_W1, r_mlp_b1, r_mlp_W2, r_mlp_b2,
           d_bias, r_bias):
    B = d_feat.shape[0]
    G = 2 * B
    k = nnidxs.shape[2]
    ns = sidxs.shape[1]

    x = jnp.concatenate([d_feat.transpose(0, 2, 3, 1),
                         r_feat.transpose(0, 2, 3, 1)], axis=0)
    x_pad = _pad_hw(x)
    m = masks.astype(jnp.float32).transpose(0, 2, 3, 1)
    m1 = jnp.concatenate([m, m], axis=0)

    w01 = jnp.stack([
        jnp.concatenate([_conv_w(d_conv0_W), _conv_w(d_conv1_W)], axis=1),
        jnp.concatenate([_conv_w(r_conv0_W), _conv_w(r_conv1_W)], axis=1),
    ])
    b01 = jnp.stack([jnp.concatenate([d_conv0_b, d_conv1_b])[None, :],
                     jnp.concatenate([r_conv0_b, r_conv1_b])[None, :]])

    y01, y0m = _conv01(x_pad, m1, w01, b01)
    feat0 = y01[..., :_C]
    feat1 = y01[..., _C:]

    f0_flat = feat0.reshape(G * _HW, _C)
    f0m_flat = y0m.reshape(G * _HW, _C)
    sidx32 = sidxs.astype(jnp.int32)
    nn_flat = nnidxs.astype(jnp.int32).reshape(B, ns * k)
    spts = jnp.pad(spoints.transpose(0, 2, 1),
                   ((0, 0), (0, 0), (0, 13))).reshape(B * ns, 16)

    (d_nn, r_nn, p_nn, sfd, sfr, msfd, msfr) = _sc_gather(
        f0_flat, f0m_flat, sidx32, nn_flat, spts)

    w1 = jnp.zeros((144, 256), jnp.float32)
    w1 = w1.at[:131, :65].set(d_mlp_W1.T).at[:131, 128:193].set(r_mlp_W1.T)
    b1 = jnp.zeros((1, 256), jnp.float32)
    b1 = b1.at[0, :65].set(d_mlp_b1).at[0, 128:193].set(r_mlp_b1)
    w2 = jnp.zeros((1, 256), jnp.float32)
    w2 = w2.at[0, :65].set(d_mlp_W2[0]).at[0, 128:193].set(r_mlp_W2[0])

    d_rows, r_rows = _attn(
        d_nn.reshape(B * ns, k, _C), r_nn.reshape(B * ns, k, _C),
        p_nn.reshape(B * ns, k, 16), sfd, sfr, spts, msfd, msfr,
        w1, b1, w2, d_bias[None, :], r_bias[None, :])

    iot = jnp.arange(ns, dtype=jnp.int32)
    maxi = jax.vmap(lambda i: jnp.zeros((_HW,), jnp.int32).at[i].max(iot))(
        sidx32)
    winof = jax.vmap(lambda m, i: m[i])(maxi, sidx32)

    vals = jnp.concatenate([d_rows, r_rows], axis=0)
    upd_flat = _sc_scatter(f0m_flat, vals, sidx32, winof)
    upd = upd_flat.reshape(G, _H, _W, _C)

    w2 = jnp.stack([_conv_w(d_conv2_W), _conv_w(r_conv2_W)])
    b2 = jnp.stack([d_conv2_b[None, :], r_conv2_b[None, :]])
    y = _conv2(_pad_hw(upd), feat1, w2, b2)

    out = y.transpose(0, 3, 1, 2)
    return out[:B], out[B:]


# speedup vs baseline: 3.8329x; 1.5968x over previous
import functools

import jax
import jax.numpy as jnp
from jax import lax
from jax.experimental import pallas as pl
from jax.experimental.pallas import tpu as pltpu
from jax.experimental.pallas import tpu_sc as plsc

_H = 224
_W = 224
_HW = _H * _W
_C = 64
_NS = 4096
_K = 9
_NW = 32
_NNB = _NS * _K


def _conv_w(w):
    return w.transpose(2, 3, 1, 0).reshape(9 * w.shape[1], w.shape[0])


_TR = 8


def _patches(x0, x1):
    x = jnp.concatenate([x0, x1], axis=0)
    cols = []
    for dy in range(3):
        for dx in range(3):
            cols.append(x[dy:dy + _TR, dx:dx + _W, :])
    p = jnp.concatenate(cols, axis=2)
    return p.reshape(_TR * _W, 576)


def _convA_body(x0, x1, m, w, b, out, outm):
    p = _patches(x0[0], x1[0])
    y = jnp.dot(p, w[0], preferred_element_type=jnp.float32) + b[0]
    y0 = jax.nn.relu(y[:, :_C])
    out[0] = jnp.concatenate([y0, y[:, _C:]], axis=1).reshape(_TR, _W, 2 * _C)
    outm[0] = y0.reshape(_TR, _W, _C) * (1.0 - m[0])


def _convE_body(x0, x1, f1, w, b, out):
    p = _patches(x0[0], x1[0])
    y = jnp.dot(p, w[0], preferred_element_type=jnp.float32) + b[0]
    out[0] = jax.nn.relu(y.reshape(_TR, _W, _C) + f1[0])


def _row_specs():
    def mk(dy):
        return pl.BlockSpec((1, _TR, _W + 2, _C),
                            lambda g, i, dy=dy: (g, i + dy, 0, 0))
    return [mk(0), mk(1)]


def _conv01(x_pad, m1, w01, b01):
    G = x_pad.shape[0]
    grid = (G, _H // _TR)
    in_specs = _row_specs() + [
        pl.BlockSpec((1, _TR, _W, 1), lambda g, i: (g, i, 0, 0)),
        pl.BlockSpec((1, 576, 2 * _C), lambda g, i: (g // 2, 0, 0)),
        pl.BlockSpec((1, 1, 2 * _C), lambda g, i: (g // 2, 0, 0)),
    ]
    out_specs = [
        pl.BlockSpec((1, _TR, _W, 2 * _C), lambda g, i: (g, i, 0, 0)),
        pl.BlockSpec((1, _TR, _W, _C), lambda g, i: (g, i, 0, 0)),
    ]
    return pl.pallas_call(
        _convA_body,
        grid=grid,
        in_specs=in_specs,
        out_specs=out_specs,
        out_shape=[
            jax.ShapeDtypeStruct((G, _H, _W, 2 * _C), jnp.float32),
            jax.ShapeDtypeStruct((G, _H, _W, _C), jnp.float32),
        ],
        compiler_params=pltpu.CompilerParams(
            dimension_semantics=("parallel", "arbitrary")),
    )(x_pad, x_pad, m1, w01, b01)


def _conv2(x_pad, f1, w2, b2):
    G = x_pad.shape[0]
    grid = (G, _H // _TR)
    in_specs = _row_specs() + [
        pl.BlockSpec((1, _TR, _W, _C), lambda g, i: (g, i, 0, 0)),
        pl.BlockSpec((1, 576, _C), lambda g, i: (g // 2, 0, 0)),
        pl.BlockSpec((1, 1, _C), lambda g, i: (g // 2, 0, 0)),
    ]
    out_specs = pl.BlockSpec((1, _TR, _W, _C), lambda g, i: (g, i, 0, 0))
    return pl.pallas_call(
        _convE_body,
        grid=grid,
        in_specs=in_specs,
        out_specs=out_specs,
        out_shape=jax.ShapeDtypeStruct((G, _H, _W, _C), jnp.float32),
        compiler_params=pltpu.CompilerParams(
            dimension_semantics=("parallel", "arbitrary")),
    )(x_pad, x_pad, f1, w2, b2)


def _pad_hw(x):
    return jnp.pad(x, ((0, 0), (1, 7), (1, 1), (0, 0)))


def _sc_gather(f0, f0m, sidxs, nnidx, spts):
    mesh = plsc.VectorSubcoreMesh(core_axis_name="c", subcore_axis_name="s")
    B = sidxs.shape[0]
    nchunk = _NNB // _NW // 128

    out_type = [
        jax.ShapeDtypeStruct((B * _NNB, _C), jnp.float32),
        jax.ShapeDtypeStruct((B * _NNB, _C), jnp.float32),
        jax.ShapeDtypeStruct((B * _NNB, 16), jnp.float32),
        jax.ShapeDtypeStruct((B * _NS, _C), jnp.float32),
        jax.ShapeDtypeStruct((B * _NS, _C), jnp.float32),
        jax.ShapeDtypeStruct((B * _NS, _C), jnp.float32),
        jax.ShapeDtypeStruct((B * _NS, _C), jnp.float32),
    ]

    @functools.partial(
        pl.kernel, mesh=mesh, out_type=out_type,
        compiler_params=pltpu.CompilerParams(needs_layout_passes=False, use_tc_tiling_on_sc=False),
        scratch_types=[
            pltpu.VMEM((_NS,), jnp.int32),
            pltpu.VMEM((2, 128), jnp.int32),
            pltpu.VMEM((2, 128), jnp.int32),
            pltpu.VMEM((2, 128), jnp.int32),
            pltpu.VMEM((2, 128), jnp.int32),
            pltpu.VMEM((2, 128, _C), jnp.float32),
            pltpu.VMEM((2, 128, _C), jnp.float32),
            pltpu.VMEM((2, 128, 16), jnp.float32),
            pltpu.SemaphoreType.DMA((2, 3)),
            pltpu.SemaphoreType.DMA((2, 3)),
        ],
    )
    def body(f0_h, f0m_h, sidx_h, nn_h, sp_h,
             dnn_h, rnn_h, pnn_h, sfd_h, sfr_h, msfd_h, msfr_h,
             sidx_v, nnq_v, idxd_v, idxr_v, idxp_v, bufd, bufr, bufp,
             gsems, osems):
        wid = lax.axis_index("s") * 2 + lax.axis_index("c")
        for b in range(B):
            offd = b * _HW
            offr = (2 + b) * _HW
            pltpu.sync_copy(sidx_h.at[b], sidx_v)

            gathers = [None, None]

            def stage(j, p):
                q = wid * nchunk + j
                row0 = q * 128
                pltpu.sync_copy(nn_h.at[b, pl.ds(row0, 128)], nnq_v.at[p])

                def compose(t, _):
                    i16 = nnq_v[p, pl.ds(t * 16, 16)]
                    s16 = plsc.load_gather(sidx_v, [i16])
                    idxd_v[p, pl.ds(t * 16, 16)] = s16 + offd
                    idxr_v[p, pl.ds(t * 16, 16)] = s16 + offr
                    idxp_v[p, pl.ds(t * 16, 16)] = i16 + b * _NS
                    return 0
                lax.fori_loop(0, 8, compose, 0)
                return (
                    pltpu.async_copy(f0_h.at[idxd_v.at[p]], bufd.at[p],
                                     gsems.at[p, 0]),
                    pltpu.async_copy(f0_h.at[idxr_v.at[p]], bufr.at[p],
                                     gsems.at[p, 1]),
                    pltpu.async_copy(sp_h.at[idxp_v.at[p]], bufp.at[p],
                                     gsems.at[p, 2]),
                )

            outs = [None, None]

            def writeback(j, p):
                for c in gathers[p]:
                    c.wait()
                o0 = b * _NNB + (wid * nchunk + j) * 128
                outs[p] = (
                    pltpu.async_copy(bufd.at[p], dnn_h.at[pl.ds(o0, 128)],
                                     osems.at[p, 0]),
                    pltpu.async_copy(bufr.at[p], rnn_h.at[pl.ds(o0, 128)],
                                     osems.at[p, 1]),
                    pltpu.async_copy(bufp.at[p], pnn_h.at[pl.ds(o0, 128)],
                                     osems.at[p, 2]),
                )

            for j in range(nchunk):
                p = j % 2
                if j >= 2:
                    for c in outs[p]:
                        c.wait()
                gathers[p] = stage(j, p)
                if j >= 1:
                    writeback(j - 1, 1 - p)
            writeback(nchunk - 1, (nchunk - 1) % 2)
            for pp in (0, 1):
                for c in outs[pp]:
                    c.wait()

            def compose_s(t, _):
                s16 = sidx_v[pl.ds(wid * 128 + t * 16, 16)]
                idxd_v[0, pl.ds(t * 16, 16)] = s16 + offd
                idxr_v[0, pl.ds(t * 16, 16)] = s16 + offr
                return 0
            lax.fori_loop(0, 8, compose_s, 0)
            o0 = b * _NS + wid * 128
            c1 = pltpu.async_copy(f0_h.at[idxd_v.at[0]], bufd.at[0],
                                  gsems.at[0, 0])
            c2 = pltpu.async_copy(f0_h.at[idxr_v.at[0]], bufr.at[0],
                                  gsems.at[0, 1])
            c3 = pltpu.async_copy(f0m_h.at[idxd_v.at[0]], bufd.at[1],
                                  gsems.at[1, 0])
            c4 = pltpu.async_copy(f0m_h.at[idxr_v.at[0]], bufr.at[1],
                                  gsems.at[1, 1])
            c1.wait(); c2.wait(); c3.wait(); c4.wait()
            w1 = pltpu.async_copy(bufd.at[0], sfd_h.at[pl.ds(o0, 128)],
                                  osems.at[0, 0])
            w2 = pltpu.async_copy(bufr.at[0], sfr_h.at[pl.ds(o0, 128)],
                                  osems.at[0, 1])
            w3 = pltpu.async_copy(bufd.at[1], msfd_h.at[pl.ds(o0, 128)],
                                  osems.at[1, 0])
            w4 = pltpu.async_copy(bufr.at[1], msfr_h.at[pl.ds(o0, 128)],
                                  osems.at[1, 1])
            w1.wait(); w2.wait(); w3.wait(); w4.wait()

    return body(f0, f0m, sidxs, nnidx, spts)


def _sc_scatter(base, vals, sidxs, winof):
    mesh = plsc.VectorSubcoreMesh(core_axis_name="c", subcore_axis_name="s")
    B = sidxs.shape[0]
    samp_per_tile = _NS // 16

    def stateful(refs):
        out_h, vals_h, sidx_h, win_h = refs

        @pl.core_map(
            mesh,
            compiler_params=pltpu.CompilerParams(
                needs_layout_passes=False, use_tc_tiling_on_sc=False),
            scratch_shapes=[
                pltpu.VMEM((128,), jnp.int32),
                pltpu.VMEM((128,), jnp.int32),
                pltpu.VMEM((128,), jnp.int32),
                pltpu.VMEM((128,), jnp.int32),
                pltpu.VMEM((128, _C), jnp.float32),
                pltpu.SemaphoreType.DMA,
            ],
        )
        def _(sidq_v, winq_v, widx_v, tidx_v, bufv, sem):
            cid = lax.axis_index("c")
            sid = lax.axis_index("s")
            for b in range(B):
                voff = cid * (B * _NS) + b * _NS
                toff = (2 * cid + b) * _HW
                for u in range(samp_per_tile // 128):
                    s0 = sid * samp_per_tile + u * 128
                    pltpu.sync_copy(sidx_h.at[b, pl.ds(s0, 128)], sidq_v)
                    pltpu.sync_copy(win_h.at[b, pl.ds(s0, 128)], winq_v)

                    def compose(t, _):
                        widx_v[pl.ds(t * 16, 16)] = (
                            winq_v[pl.ds(t * 16, 16)] + voff)
                        tidx_v[pl.ds(t * 16, 16)] = (
                            sidq_v[pl.ds(t * 16, 16)] + toff)
                        return 0
                    lax.fori_loop(0, 8, compose, 0)
                    pltpu.async_copy(vals_h.at[widx_v], bufv, sem).wait()
                    pltpu.async_copy(bufv, out_h.at[tidx_v], sem).wait()

    out, _, _, _ = pl.run_state(stateful)((base, vals, sidxs, winof))
    return out


def _attn_body(dnn, rnn, pnn, sfd, sfr, sp, msfd, msfr,
               w1, b1, w2, bd, br, outd, outr):
    d3 = dnn[...]
    r3 = rnn[...]
    p3 = pnn[...]
    xs = jnp.concatenate([sfd[...], sfr[...], sp[...]], axis=1)
    base = jnp.dot(xs, w1[...], preferred_element_type=jnp.float32)
    dlogs, rlogs = [], []
    for kk in range(_K):
        xk = jnp.concatenate([d3[:, kk, :], r3[:, kk, :], p3[:, kk, :]],
                             axis=1)
        g = jnp.dot(xk, w1[...], preferred_element_type=jnp.float32)
        pre = g - base + b1[...]
        h = jnp.where(pre >= 0, pre, 0.2 * pre)
        prod = h * w2[...]
        dlogs.append(jnp.sum(prod[:, :128], axis=1, keepdims=True))
        rlogs.append(jnp.sum(prod[:, 128:], axis=1, keepdims=True))
    dlog = jnp.concatenate(dlogs, axis=1)
    rlog = jnp.concatenate(rlogs, axis=1)

    def soft(x):
        m = jnp.max(x, axis=1, keepdims=True)
        e = jnp.exp(x - m)
        return e / jnp.sum(e, axis=1, keepdims=True)
    dattn = soft(dlog)
    rattn = soft(rlog)
    accd = msfd[...] + bd[...]
    accr = msfr[...] + br[...]
    for kk in range(_K):
        accd = accd + dattn[:, kk:kk + 1] * d3[:, kk, :]
        accr = accr + rattn[:, kk:kk + 1] * r3[:, kk, :]
    outd[...] = accd
    outr[...] = accr


def _attn(dnn3, rnn3, pnn3, sfd, sfr, sp, msfd, msfr, w1, b1, w2, bd, br):
    S = sfd.shape[0]
    SB = 512
    grid = (S // SB,)
    bs2 = lambda c: pl.BlockSpec((SB, c), lambda i: (i, 0))
    bs3 = lambda c: pl.BlockSpec((SB, _K, c), lambda i: (i, 0, 0))
    full = lambda a, b: pl.BlockSpec((a, b), lambda i: (0, 0))
    return pl.pallas_call(
        _attn_body,
        grid=grid,
        in_specs=[bs3(_C), bs3(_C), bs3(16), bs2(_C), bs2(_C), bs2(16),
                  bs2(_C), bs2(_C),
                  full(144, 256), full(1, 256), full(1, 256),
                  full(1, _C), full(1, _C)],
        out_specs=[bs2(_C), bs2(_C)],
        out_shape=[jax.ShapeDtypeStruct((S, _C), jnp.float32),
                   jax.ShapeDtypeStruct((S, _C), jnp.float32)],
        compiler_params=pltpu.CompilerParams(
            dimension_semantics=("arbitrary",)),
    )(dnn3, rnn3, pnn3, sfd, sfr, sp, msfd, msfr, w1, b1, w2, bd, br)


def kernel(d_feat, r_feat, spoints, sidxs, nnidxs, masks, nsamples,
           d_conv0_W, d_conv0_b, d_conv1_W, d_conv1_b, d_conv2_W, d_conv2_b,
           r_conv0_W, r_conv0_b, r_conv1_W, r_conv1_b, r_conv2_W, r_conv2_b,
           d_mlp_W1, d_mlp_b1, d_mlp_W2, d_mlp_b2,
           r_mlp_W1, r_mlp_b1, r_mlp_W2, r_mlp_b2,
           d_bias, r_bias):
    B = d_feat.shape[0]
    G = 2 * B
    k = nnidxs.shape[2]
    ns = sidxs.shape[1]

    x = jnp.concatenate([d_feat.transpose(0, 2, 3, 1),
                         r_feat.transpose(0, 2, 3, 1)], axis=0)
    x_pad = _pad_hw(x)
    m = masks.astype(jnp.float32).transpose(0, 2, 3, 1)
    m1 = jnp.concatenate([m, m], axis=0)

    w01 = jnp.stack([
        jnp.concatenate([_conv_w(d_conv0_W), _conv_w(d_conv1_W)], axis=1),
        jnp.concatenate([_conv_w(r_conv0_W), _conv_w(r_conv1_W)], axis=1),
    ])
    b01 = jnp.stack([jnp.concatenate([d_conv0_b, d_conv1_b])[None, :],
                     jnp.concatenate([r_conv0_b, r_conv1_b])[None, :]])

    y01, y0m = _conv01(x_pad, m1, w01, b01)
    feat0 = y01[..., :_C]
    feat1 = y01[..., _C:]

    f0_flat = feat0.reshape(G * _HW, _C)
    f0m_flat = y0m.reshape(G * _HW, _C)
    sidx32 = sidxs.astype(jnp.int32)
    nn_flat = nnidxs.astype(jnp.int32).reshape(B, ns * k)
    spts = jnp.pad(spoints.transpose(0, 2, 1),
                   ((0, 0), (0, 0), (0, 13))).reshape(B * ns, 16)

    (d_nn, r_nn, p_nn, sfd, sfr, msfd, msfr) = _sc_gather(
        f0_flat, f0m_flat, sidx32, nn_flat, spts)

    w1 = jnp.zeros((144, 256), jnp.float32)
    w1 = w1.at[:131, :65].set(d_mlp_W1.T).at[:131, 128:193].set(r_mlp_W1.T)
    b1 = jnp.zeros((1, 256), jnp.float32)
    b1 = b1.at[0, :65].set(d_mlp_b1).at[0, 128:193].set(r_mlp_b1)
    w2 = jnp.zeros((1, 256), jnp.float32)
    w2 = w2.at[0, :65].set(d_mlp_W2[0]).at[0, 128:193].set(r_mlp_W2[0])

    d_rows, r_rows = _attn(
        d_nn.reshape(B * ns, k, _C), r_nn.reshape(B * ns, k, _C),
        p_nn.reshape(B * ns, k, 16), sfd, sfr, spts, msfd, msfr,
        w1, b1, w2, d_bias[None, :], r_bias[None, :])

    iot = jnp.arange(ns, dtype=jnp.int32)
    maxi = jax.vmap(lambda i: jnp.zeros((_HW,), jnp.int32).at[i].max(iot))(
        sidx32)
    winof = jax.vmap(lambda m, i: m[i])(maxi, sidx32)

    vals = jnp.concatenate([d_rows, r_rows], axis=0)
    upd_flat = _sc_scatter(f0m_flat, vals, sidx32, winof)
    upd = upd_flat.reshape(G, _H, _W, _C)

    w2 = jnp.stack([_conv_w(d_conv2_W), _conv_w(r_conv2_W)])
    b2 = jnp.stack([d_conv2_b[None, :], r_conv2_b[None, :]])
    y = _conv2(_pad_hw(upd), feat1, w2, b2)

    out = y.transpose(0, 3, 1, 2)
    return out[:B], out[B:]
